# initial kernel scaffold (unmeasured)
import jax
import jax.numpy as jnp
from jax import lax
from jax.experimental import pallas as pl
from jax.experimental.pallas import tpu as pltpu


def kernel(
    x,
):
    def body(*refs):
        pass

    out_shape = jax.ShapeDtypeStruct(..., jnp.float32)
    return pl.pallas_call(body, out_shape=out_shape)(...)


# baseline (device time: 110047 ns/iter reference)
import jax
import jax.numpy as jnp
from jax import lax
from jax.experimental import pallas as pl
from jax.experimental.pallas import tpu as pltpu

N_DEV = 16


def _stage(v, row_g, j, k):
    down = jnp.roll(v, j, axis=0)
    up = jnp.roll(v, -j, axis=0)
    is_lower = (row_g & j) == 0
    partner = jnp.where(is_lower, up, down)
    take_min = is_lower == ((row_g & k) == 0)
    return jnp.where(take_min, jnp.minimum(v, partner), jnp.maximum(v, partner))


def kernel(x):
    m_per, n = x.shape
    total = N_DEV * m_per
    n_rounds = 10

    def body(x_ref, out_ref, block_ref, recv_ref, send_sems, recv_sems):
        me = lax.axis_index("i")

        partners = [jnp.bitwise_xor(me, 1 << b) for b in range(4)]
        barrier_sem = pltpu.get_barrier_semaphore()
        for p in partners:
            pl.semaphore_signal(
                barrier_sem, inc=1,
                device_id=(p,), device_id_type=pl.DeviceIdType.MESH,
            )
        pl.semaphore_wait(barrier_sem, len(partners))

        block_ref[:, :] = x_ref[:, :].astype(jnp.bfloat16)
        row_g = lax.broadcasted_iota(jnp.int32, (m_per, 1), 0) + me * m_per

        r = 0
        k = 2
        while k <= total:
            j = k // 2
            while j >= 1:
                if j >= m_per:
                    jb = j // m_per
                    kb = k // m_per
                    partner = jnp.bitwise_xor(me, jb)
                    rdma = pltpu.make_async_remote_copy(
                        src_ref=block_ref,
                        dst_ref=recv_ref.at[r % 2],
                        send_sem=send_sems.at[r],
                        recv_sem=recv_sems.at[r],
                        device_id=(partner,),
                        device_id_type=pl.DeviceIdType.MESH,
                    )
                    rdma.start()
                    rdma.wait()
                    a = block_ref[:, :]
                    b = recv_ref[r % 2, :, :]
                    take_min = jnp.logical_xor(
                        (me & jb) != 0, (me & kb) == 0
                    )
                    block_ref[:, :] = jnp.where(
                        take_min, jnp.minimum(a, b), jnp.maximum(a, b)
                    )
                    r += 1
                else:
                    block_ref[:, :] = _stage(block_ref[:, :], row_g, j, k)
                j //= 2
            k *= 2
        assert r == n_rounds

        out_ref[:, :] = block_ref[:, :].astype(jnp.float32)

    return pl.pallas_call(
        body,
        out_shape=jax.ShapeDtypeStruct((m_per, n), jnp.float32),
        in_specs=[pl.BlockSpec(memory_space=pltpu.VMEM)],
        out_specs=pl.BlockSpec(memory_space=pltpu.VMEM),
        scratch_shapes=[
            pltpu.VMEM((m_per, n), jnp.bfloat16),
            pltpu.VMEM((2, m_per, n), jnp.bfloat16),
            pltpu.SemaphoreType.DMA((n_rounds,)),
            pltpu.SemaphoreType.DMA((n_rounds,)),
        ],
        compiler_params=pltpu.CompilerParams(collective_id=0),
    )(x)
